# baseline (speedup 1.0000x reference)
import jax
import jax.numpy as jnp
from jax import lax
from jax.experimental import pallas as pl
from jax.experimental.pallas import tpu as pltpu
from jax.experimental.pallas import tpu_sc as plsc

N_NODES = 10000
N_CLIQUES = 5000
N_EDGES = 320000
D = 128

NC = 2
NS = 16
NW = NC * NS
L = 16
CH = 64
NCHUNK = 160
NSLOT = 4
NGROUP = NCHUNK // NSLOT
E_PER = NCHUNK * CH
E_PAD = NW * E_PER
TRASH = 5119
C_PAD = 5120
ROWS_PER_TILE = C_PAD // NS
ZR = 32


def _sc_body(x_hbm, nidx_hbm, cidx_hbm, zeros_hbm,
             psum_hbm, pcnt_hbm,
             nidx_all, cidx_all,
             nrow0, crow0, nrow1, crow1, nrow2, crow2, nrow3, crow3,
             buf0, buf1, buf2, buf3, cnt_v, zv,
             acc, sg0, sg1, sg2, sg3, ss0, ss1, ss2, ss3):
    nrows = [nrow0, nrow1, nrow2, nrow3]
    crows = [crow0, crow1, crow2, crow3]
    bufs = [buf0, buf1, buf2, buf3]
    sgs = [sg0, sg1, sg2, sg3]
    sss = [ss0, ss1, ss2, ss3]

    cid = lax.axis_index("c")
    sid = lax.axis_index("s")
    wid = cid * NS + sid

    pltpu.sync_copy(nidx_hbm.at[wid], nidx_all)
    pltpu.sync_copy(cidx_hbm.at[wid], cidx_all)
    pltpu.sync_copy(zeros_hbm, zv)

    z16 = jnp.zeros((L,), jnp.float32)

    @pl.loop(0, C_PAD // L)
    def _zero_cnt(i):
        cnt_v[pl.ds(i * L, L)] = z16

    r0 = sid * ROWS_PER_TILE
    for z in range(ROWS_PER_TILE // ZR):
        pltpu.sync_copy(zv, acc.at[pl.ds(r0 + z * ZR, ZR)])
    plsc.subcore_barrier()

    one16 = jnp.ones((L,), jnp.float32)

    xoff = cid * N_NODES

    def load_rows(j, nr, cr):
        for k in range(CH // L):
            nr[pl.ds(k * L, L)] = nidx_all[j, pl.ds(k * L, L)] + xoff
            cr[pl.ds(k * L, L)] = cidx_all[j, pl.ds(k * L, L)]

    def counts(j):
        for k in range(CH // L):
            idx = cidx_all[j, pl.ds(k * L, L)]
            plsc.addupdate_scatter(cnt_v, [idx], one16)

    def gather_start(s):
        pltpu.async_copy(x_hbm.at[nrows[s]], bufs[s], sgs[s])

    def gather_wait(s):
        pltpu.make_async_copy(x_hbm.at[nrows[s]], bufs[s], sgs[s]).wait()

    def scatter_start(s):
        pltpu.async_copy(bufs[s], acc.at[crows[s]], sss[s], add=True)

    def scatter_wait(s):
        pltpu.make_async_copy(bufs[s], acc.at[crows[s]], sss[s]).wait()

    for s in range(NSLOT):
        load_rows(s, nrows[s], crows[s])
        gather_start(s)

    def loop_body(g, _):
        j = g * NSLOT
        for s in range(NSLOT):
            gather_wait(s)
            scatter_start(s)
            counts(j + s)
        for s in range(NSLOT):
            scatter_wait(s)
            load_rows(j + NSLOT + s, nrows[s], crows[s])
            gather_start(s)
        return 0

    lax.fori_loop(0, NGROUP - 1, loop_body, 0)

    jlast = (NGROUP - 1) * NSLOT
    for s in range(NSLOT):
        gather_wait(s)
        scatter_start(s)
        counts(jlast + s)
    for s in range(NSLOT):
        scatter_wait(s)

    plsc.subcore_barrier()

    pltpu.sync_copy(acc.at[pl.ds(r0, ROWS_PER_TILE)],
                    psum_hbm.at[cid, pl.ds(r0, ROWS_PER_TILE)])
    pltpu.sync_copy(cnt_v, pcnt_hbm.at[wid])


@jax.jit
def _sc_segment_sum(x, nidx, cidx, zeros):
    mesh = plsc.VectorSubcoreMesh(core_axis_name="c", subcore_axis_name="s",
                                  num_cores=NC, num_subcores=NS)
    return pl.kernel(
        _sc_body,
        out_type=[
            jax.ShapeDtypeStruct((NC, C_PAD, D), jnp.float32),
            jax.ShapeDtypeStruct((NW, C_PAD), jnp.float32),
        ],
        mesh=mesh,
        compiler_params=pltpu.CompilerParams(needs_layout_passes=False),
        scratch_types=[
            pltpu.VMEM((NCHUNK, CH), jnp.int32),
            pltpu.VMEM((NCHUNK, CH), jnp.int32),
        ] + [pltpu.VMEM((CH,), jnp.int32) for _ in range(2 * NSLOT)] + [
            pltpu.VMEM((CH, D), jnp.float32) for _ in range(NSLOT)] + [
            pltpu.VMEM((C_PAD,), jnp.float32),
            pltpu.VMEM((ZR, D), jnp.float32),
            pltpu.VMEM_SHARED((C_PAD, D), jnp.float32),
        ] + [pltpu.SemaphoreType.DMA for _ in range(2 * NSLOT)],
    )(x, nidx, cidx, zeros)


def _tc_body(psum_ref, pcnt_ref, w_ref, b_ref, out_ref):
    s = psum_ref[0] + psum_ref[1]
    c = jnp.sum(pcnt_ref[...], axis=1, keepdims=True)
    mean = s / jnp.maximum(c, 1.0)
    out_ref[...] = lax.dot_general(
        mean, w_ref[...], (((1,), (1,)), ((), ())),
        preferred_element_type=jnp.float32) + b_ref[...]


@jax.jit
def _tc_finish(psum, pcnt_t, W, b2d):
    return pl.pallas_call(
        _tc_body,
        out_shape=jax.ShapeDtypeStruct((C_PAD, D), jnp.float32),
    )(psum, pcnt_t, W, b2d)


def kernel(x, x_clique, node2clique_index, W, b):
    pad = E_PAD - N_EDGES
    nidx = jnp.concatenate(
        [node2clique_index[0], jnp.zeros((pad,), jnp.int32)]
    ).reshape(NW, NCHUNK, CH)
    cidx = jnp.concatenate(
        [node2clique_index[1], jnp.full((pad,), TRASH, jnp.int32)]
    ).reshape(NW, NCHUNK, CH)
    zeros = jnp.zeros((ZR, D), jnp.float32)
    xx = jnp.concatenate([x, x], axis=0)
    psum, pcnt = _sc_segment_sum(xx, nidx, cidx, zeros)
    out = _tc_finish(psum, pcnt.T, W, b.reshape(1, D))
    return out[:N_CLIQUES]


# speedup vs baseline: 1.3830x; 1.3830x over previous
import jax
import jax.numpy as jnp
from jax import lax
from jax.experimental import pallas as pl
from jax.experimental.pallas import tpu as pltpu
from jax.experimental.pallas import tpu_sc as plsc

N_NODES = 10000
N_CLIQUES = 5000
N_EDGES = 320000
D = 128

NC = 2
NS = 16
NW = NC * NS
L = 16
CH = 64
NSLOT = 4
E_PAD = 327680
TOTCH = E_PAD // CH
A_CH = 248
B_CH = 72
TRASH = 5119
C_PAD = 5120
ROWS_PER_TILE = C_PAD // NS
ZR = 32


def _sc_body(x_hbm, nidx_hbm, cidx_hbm, zeros_hbm,
             psum_hbm, pcnt_hbm,
             nrow0, crow0, nrow1, crow1, nrow2, crow2, nrow3, crow3,
             buf0, buf1, buf2, buf3, cnt_v, zv,
             acc, sg0, sg1, sg2, sg3, ss0, ss1, ss2, ss3,
             si0, si1, si2, si3):
    nrows = [nrow0, nrow1, nrow2, nrow3]
    crows = [crow0, crow1, crow2, crow3]
    bufs = [buf0, buf1, buf2, buf3]
    sgs = [sg0, sg1, sg2, sg3]
    sss = [ss0, ss1, ss2, ss3]
    sis = [si0, si1, si2, si3]

    cid = lax.axis_index("c")
    sid = lax.axis_index("s")
    wid = cid * NS + sid

    pltpu.sync_copy(zeros_hbm, zv)

    z16 = jnp.zeros((L,), jnp.float32)

    @pl.loop(0, C_PAD // L)
    def _zero_cnt(i):
        cnt_v[pl.ds(i * L, L)] = z16

    r0 = sid * ROWS_PER_TILE
    for z in range(ROWS_PER_TILE // ZR):
        pltpu.sync_copy(zv, acc.at[pl.ds(r0 + z * ZR, ZR)])
    plsc.subcore_barrier()

    one16 = jnp.ones((L,), jnp.float32)

    def idx_start(ch, s):
        pltpu.async_copy(nidx_hbm.at[ch], nrows[s], sis[s])
        pltpu.async_copy(cidx_hbm.at[ch], crows[s], sis[s])

    def idx_wait(s):
        pltpu.make_async_copy(nidx_hbm.at[0], nrows[s], sis[s]).wait()
        pltpu.make_async_copy(cidx_hbm.at[0], crows[s], sis[s]).wait()

    def gather_start(s):
        pltpu.async_copy(x_hbm.at[nrows[s]], bufs[s], sgs[s])

    def gather_wait(s):
        pltpu.make_async_copy(x_hbm.at[nrows[s]], bufs[s], sgs[s]).wait()

    def scatter_start(s):
        pltpu.async_copy(bufs[s], acc.at[crows[s]], sss[s], add=True)

    def scatter_wait(s):
        pltpu.make_async_copy(bufs[s], acc.at[crows[s]], sss[s]).wait()

    def counts(s):
        for k in range(CH // L):
            idx = crows[s][pl.ds(k * L, L)]
            plsc.addupdate_scatter(cnt_v, [idx], one16)

    def pipeline(base, ngroup):
        for s in range(NSLOT):
            idx_start(base + s, s)

        def loop_body(g, _):
            j = base + g * NSLOT
            for s in range(NSLOT):
                idx_wait(s)
                gather_start(s)
            for s in range(NSLOT):
                gather_wait(s)
                scatter_start(s)
                counts(s)
            for s in range(NSLOT):
                scatter_wait(s)
                idx_start(j + NSLOT + s, s)
            return 0

        lax.fori_loop(0, ngroup - 1, loop_body, 0)

        for s in range(NSLOT):
            idx_wait(s)
            gather_start(s)
        for s in range(NSLOT):
            gather_wait(s)
            scatter_start(s)
            counts(s)
        for s in range(NSLOT):
            scatter_wait(s)

    @pl.when(cid == 0)
    def _fast_core():
        pipeline(sid * A_CH, A_CH // NSLOT)

    @pl.when(cid == 1)
    def _slow_core():
        pipeline(NS * A_CH + sid * B_CH, B_CH // NSLOT)

    plsc.subcore_barrier()

    pltpu.sync_copy(acc.at[pl.ds(r0, ROWS_PER_TILE)],
                    psum_hbm.at[cid, pl.ds(r0, ROWS_PER_TILE)])
    pltpu.sync_copy(cnt_v, pcnt_hbm.at[wid])


@jax.jit
def _sc_segment_sum(x, nidx, cidx, zeros):
    mesh = plsc.VectorSubcoreMesh(core_axis_name="c", subcore_axis_name="s",
                                  num_cores=NC, num_subcores=NS)
    return pl.kernel(
        _sc_body,
        out_type=[
            jax.ShapeDtypeStruct((NC, C_PAD, D), jnp.float32),
            jax.ShapeDtypeStruct((NW, C_PAD), jnp.float32),
        ],
        mesh=mesh,
        compiler_params=pltpu.CompilerParams(needs_layout_passes=False),
        scratch_types=[
            pltpu.VMEM((CH,), jnp.int32) for _ in range(2 * NSLOT)
        ] + [
            pltpu.VMEM((CH, D), jnp.float32) for _ in range(NSLOT)
        ] + [
            pltpu.VMEM((C_PAD,), jnp.float32),
            pltpu.VMEM((ZR, D), jnp.float32),
            pltpu.VMEM_SHARED((C_PAD, D), jnp.float32),
        ] + [pltpu.SemaphoreType.DMA for _ in range(3 * NSLOT)],
    )(x, nidx, cidx, zeros)


def _tc_body(psum_ref, pcnt_ref, w_ref, b_ref, out_ref):
    s = psum_ref[0] + psum_ref[1]
    c = jnp.sum(pcnt_ref[...], axis=1, keepdims=True)
    mean = s / jnp.maximum(c, 1.0)
    out_ref[...] = lax.dot_general(
        mean, w_ref[...], (((1,), (1,)), ((), ())),
        preferred_element_type=jnp.float32) + b_ref[...]


@jax.jit
def _tc_finish(psum, pcnt_t, W, b2d):
    return pl.pallas_call(
        _tc_body,
        out_shape=jax.ShapeDtypeStruct((C_PAD, D), jnp.float32),
    )(psum, pcnt_t, W, b2d)


def kernel(x, x_clique, node2clique_index, W, b):
    pad = E_PAD - N_EDGES
    nidx = jnp.concatenate(
        [node2clique_index[0], jnp.zeros((pad,), jnp.int32)]
    ).reshape(TOTCH, CH)
    cidx = jnp.concatenate(
        [node2clique_index[1], jnp.full((pad,), TRASH, jnp.int32)]
    ).reshape(TOTCH, CH)
    zeros = jnp.zeros((ZR, D), jnp.float32)
    psum, pcnt = _sc_segment_sum(x, nidx, cidx, zeros)
    out = _tc_finish(psum, pcnt.T, W, b.reshape(1, D))
    return out[:N_CLIQUES]
